# Initial kernel scaffold; baseline (speedup 1.0000x reference)
import jax
import jax.numpy as jnp
from jax.experimental import pallas as pl


def kernel(x, edge_index, batch, W_nf, b_nf, W1, b1, g1, be1, W2, b2, g2, be2, Wg, bg):
    raise NotImplementedError("write your pallas kernel here")


# speedup vs baseline: 10.7131x; 10.7131x over previous
import functools

import jax
import jax.numpy as jnp
from jax import lax
from jax.experimental import pallas as pl
from jax.experimental.pallas import tpu as pltpu
from jax.experimental.pallas import tpu_sc as plsc

_N = 10000
_E = 160000
_G = 64
_CH = 128
_NT = 16
_NC = 2
_ACC_ROWS = 10240
_RPT = _ACC_ROWS // _NT


def _sc_degree(dst):
  mesh = plsc.VectorSubcoreMesh(core_axis_name="c", subcore_axis_name="s")
  main_e = 5120
  main_ch = main_e // _CH
  last_ch = (_E - 31 * main_e) // _CH

  @functools.partial(
      pl.kernel,
      out_type=jax.ShapeDtypeStruct((_NC, _ACC_ROWS, 16), jnp.float32),
      mesh=mesh,
      scratch_types=[
          pltpu.VMEM((_CH, 16), jnp.float32),
          pltpu.VMEM((_CH,), jnp.int32),
          pltpu.VMEM_SHARED((_ACC_ROWS, 16), jnp.float32),
      ],
  )
  def deg_kernel(dst_hbm, out_hbm, buf, didx, acc):
    c = lax.axis_index("c")
    s = lax.axis_index("s")
    wid = s * _NC + c

    def _zero(i, carry):
      buf[i, :] = jnp.zeros((16,), jnp.float32)
      return carry
    lax.fori_loop(0, _CH, _zero, None)
    for k in range(_RPT // _CH):
      pltpu.sync_copy(buf, acc.at[pl.ds(s * _RPT + k * _CH, _CH)])
    plsc.subcore_barrier()

    def _ones(i, carry):
      buf[i, :] = jnp.ones((16,), jnp.float32)
      return carry
    lax.fori_loop(0, _CH, _ones, None)

    nch = jnp.where(wid == 31, last_ch, main_ch)
    base = wid * main_e

    def _body(j, carry):
      pltpu.sync_copy(dst_hbm.at[pl.ds(base + j * _CH, _CH)], didx)
      pltpu.sync_copy(buf, acc.at[didx], add=True)
      return carry
    lax.fori_loop(0, nch, _body, None)
    plsc.subcore_barrier()
    r0 = s * _RPT
    pltpu.sync_copy(acc.at[pl.ds(r0, _RPT)],
                    out_hbm.at[c].at[pl.ds(r0, _RPT)])

  return deg_kernel(dst)[:, :_N, :]


def _sc_edge_agg(hs_a, hs_b, src, dst, split_edges):
  per_core = _E // 2 if split_edges else _E
  main_e = per_core // _NT - (per_core // _NT) % _CH
  main_ch = main_e // _CH
  last_ch = (per_core - 15 * main_e) // _CH
  assert 15 * main_e + last_ch * _CH == per_core
  mesh = plsc.VectorSubcoreMesh(core_axis_name="c", subcore_axis_name="s")

  @functools.partial(
      pl.kernel,
      out_type=(jax.ShapeDtypeStruct((_ACC_ROWS, 128), jnp.float32),
                jax.ShapeDtypeStruct((_ACC_ROWS, 128), jnp.float32)),
      mesh=mesh,
      scratch_types=[
          pltpu.VMEM((_CH, 128), jnp.float32),
          pltpu.VMEM((_CH,), jnp.int32),
          pltpu.VMEM((_CH,), jnp.int32),
          pltpu.VMEM_SHARED((_ACC_ROWS, 128), jnp.float32),
          pltpu.SemaphoreType.DMA,
      ],
  )
  def agg_kernel(hsa_hbm, hsb_hbm, src_hbm, dst_hbm, outa_hbm, outb_hbm,
                 rows, sidx, didx, acc, sem):
    c = lax.axis_index("c")
    s = lax.axis_index("s")

    def _zero(i, carry):
      for j in range(8):
        rows[i, pl.ds(j * 16, 16)] = jnp.zeros((16,), jnp.float32)
      return carry
    lax.fori_loop(0, _CH, _zero, None)
    for k in range(_RPT // _CH):
      pltpu.sync_copy(rows, acc.at[pl.ds(s * _RPT + k * _CH, _CH)])
    plsc.subcore_barrier()

    nch = jnp.where(s == 15, last_ch, main_ch)
    base = (c * per_core if split_edges else 0) + s * main_e

    def _run(hs_hbm):
      def _body(j, carry):
        e0 = base + j * _CH
        pltpu.sync_copy(src_hbm.at[pl.ds(e0, _CH)], sidx)
        pltpu.sync_copy(dst_hbm.at[pl.ds(e0, _CH)], didx)
        pltpu.async_copy(hs_hbm.at[sidx], rows, sem).wait()
        pltpu.sync_copy(rows, acc.at[didx], add=True)
        return carry
      lax.fori_loop(0, nch, _body, None)

    @pl.when(c == 0)
    def _():
      _run(hsa_hbm)

    @pl.when(c == 1)
    def _():
      _run(hsb_hbm)

    plsc.subcore_barrier()
    r0 = s * _RPT

    @pl.when(c == 0)
    def _():
      pltpu.sync_copy(acc.at[pl.ds(r0, _RPT)],
                      outa_hbm.at[pl.ds(r0, _RPT)])

    @pl.when(c == 1)
    def _():
      pltpu.sync_copy(acc.at[pl.ds(r0, _RPT)],
                      outb_hbm.at[pl.ds(r0, _RPT)])

  oa, ob = agg_kernel(hs_a, hs_b, src, dst)
  return oa[:_N], ob[:_N]


_B = 1000


def _tc_head(x, W_nf, b_nf, W1, cnt):
  def body(x_r, wnf_r, bnf_r, w1_r, cnt_r, hs1_r, dinv_r):
    Wf = jnp.dot(wnf_r[...], w1_r[...], preferred_element_type=jnp.float32)
    bf = jnp.dot(bnf_r[...], w1_r[...], preferred_element_type=jnp.float32)
    h1 = jnp.dot(x_r[...], Wf, preferred_element_type=jnp.float32) + bf
    deg = cnt_r[0, :, 0:1] + cnt_r[1, :, 0:1] + 1.0
    dinv = lax.rsqrt(deg)
    dinv_r[...] = dinv
    hs1_r[...] = h1 * dinv

  return pl.pallas_call(
      body,
      grid=(_N // _B,),
      in_specs=[
          pl.BlockSpec((_B, 256), lambda i: (i, 0)),
          pl.BlockSpec((256, 256), lambda i: (0, 0)),
          pl.BlockSpec((1, 256), lambda i: (0, 0)),
          pl.BlockSpec((256, 128), lambda i: (0, 0)),
          pl.BlockSpec((2, _B, 16), lambda i: (0, i, 0)),
      ],
      out_specs=[pl.BlockSpec((_B, 128), lambda i: (i, 0)),
                 pl.BlockSpec((_B, 1), lambda i: (i, 0))],
      out_shape=[jax.ShapeDtypeStruct((_N, 128), jnp.float32),
                 jax.ShapeDtypeStruct((_N, 1), jnp.float32)],
  )(x, W_nf, b_nf.reshape(1, 256), W1, cnt)


def _tc_pre1(agg_a, agg_b, hs1, dinv, b1):
  def body(aa, ab, hs, dv, b1_r, pre_r, st_r):
    pre = (aa[...] + ab[...] + hs[...]) * dv[...] + b1_r[...]
    pre_r[...] = pre

    @pl.when(pl.program_id(0) == 0)
    def _():
      st_r[...] = jnp.zeros_like(st_r)
    st_r[0:1, :] += jnp.sum(pre, axis=0, keepdims=True)
    st_r[1:2, :] += jnp.sum(pre * pre, axis=0, keepdims=True)

  return pl.pallas_call(
      body,
      grid=(_N // _B,),
      in_specs=[
          pl.BlockSpec((_B, 128), lambda i: (i, 0)),
          pl.BlockSpec((_B, 128), lambda i: (i, 0)),
          pl.BlockSpec((_B, 128), lambda i: (i, 0)),
          pl.BlockSpec((_B, 1), lambda i: (i, 0)),
          pl.BlockSpec((1, 128), lambda i: (0, 0)),
      ],
      out_specs=[pl.BlockSpec((_B, 128), lambda i: (i, 0)),
                 pl.BlockSpec((2, 128), lambda i: (0, 0))],
      out_shape=[jax.ShapeDtypeStruct((_N, 128), jnp.float32),
                 jax.ShapeDtypeStruct((2, 128), jnp.float32)],
  )(agg_a, agg_b, hs1, dinv, b1.reshape(1, 128))


def _bn_apply(pre, st, g, be):
  mu = st[0:1, :] / _N
  var = st[1:2, :] / _N - mu * mu
  xh = (pre - mu) * lax.rsqrt(var + 1e-5) * g + be
  return jnp.where(xh >= 0, xh, 0.01 * xh)


def _tc_conv2_in(pre1, st1, g1, be1, W2, dinv):
  def body(pre_r, st_r, g_r, be_r, w2_r, dv_r, ha_r, hb_r):
    d1 = _bn_apply(pre_r[...], st_r[...], g_r[...], be_r[...])
    h2 = jnp.dot(d1, w2_r[...], preferred_element_type=jnp.float32)
    hs2 = h2 * dv_r[...]
    ha_r[...] = hs2[:, :128]
    hb_r[...] = hs2[:, 128:]

  return pl.pallas_call(
      body,
      grid=(_N // _B,),
      in_specs=[
          pl.BlockSpec((_B, 128), lambda i: (i, 0)),
          pl.BlockSpec((2, 128), lambda i: (0, 0)),
          pl.BlockSpec((1, 128), lambda i: (0, 0)),
          pl.BlockSpec((1, 128), lambda i: (0, 0)),
          pl.BlockSpec((128, 256), lambda i: (0, 0)),
          pl.BlockSpec((_B, 1), lambda i: (i, 0)),
      ],
      out_specs=[pl.BlockSpec((_B, 128), lambda i: (i, 0)),
                 pl.BlockSpec((_B, 128), lambda i: (i, 0))],
      out_shape=[jax.ShapeDtypeStruct((_N, 128), jnp.float32),
                 jax.ShapeDtypeStruct((_N, 128), jnp.float32)],
  )(pre1, st1, g1.reshape(1, 128), be1.reshape(1, 128), W2, dinv)


def _tc_pre2(agg_a, agg_b, hs2a, hs2b, dinv, b2):
  def body(aa, ab, ha, hb, dv, b2_r, pre_r, st_r):
    pa = (aa[...] + ha[...]) * dv[...]
    pb = (ab[...] + hb[...]) * dv[...]
    pre = jnp.concatenate([pa, pb], axis=1) + b2_r[...]
    pre_r[...] = pre

    @pl.when(pl.program_id(0) == 0)
    def _():
      st_r[...] = jnp.zeros_like(st_r)
    st_r[0:1, :] += jnp.sum(pre, axis=0, keepdims=True)
    st_r[1:2, :] += jnp.sum(pre * pre, axis=0, keepdims=True)

  return pl.pallas_call(
      body,
      grid=(_N // _B,),
      in_specs=[
          pl.BlockSpec((_B, 128), lambda i: (i, 0)),
          pl.BlockSpec((_B, 128), lambda i: (i, 0)),
          pl.BlockSpec((_B, 128), lambda i: (i, 0)),
          pl.BlockSpec((_B, 128), lambda i: (i, 0)),
          pl.BlockSpec((_B, 1), lambda i: (i, 0)),
          pl.BlockSpec((1, 256), lambda i: (0, 0)),
      ],
      out_specs=[pl.BlockSpec((_B, 256), lambda i: (i, 0)),
                 pl.BlockSpec((2, 256), lambda i: (0, 0))],
      out_shape=[jax.ShapeDtypeStruct((_N, 256), jnp.float32),
                 jax.ShapeDtypeStruct((2, 256), jnp.float32)],
  )(agg_a, agg_b, hs2a, hs2b, dinv, b2.reshape(1, 256))


def _tc_gate(pre2, st2, g2, be2, Wg, bg, batch2d):
  def body(pre_r, st_r, g_r, be_r, wg_r, bg_r, bt_r, d2_r, gate_r, m_r):
    d2 = _bn_apply(pre_r[...], st_r[...], g_r[...], be_r[...])
    d2_r[...] = d2
    gate = jnp.dot(d2, wg_r[...], preferred_element_type=jnp.float32) + bg_r[...]
    gate_r[...] = gate
    oh = bt_r[...] == lax.broadcasted_iota(jnp.int32, (1, _G), 1)
    part = jnp.max(jnp.where(oh, gate, -3e38), axis=0, keepdims=True)

    @pl.when(pl.program_id(0) == 0)
    def _():
      m_r[...] = jnp.full_like(m_r, -3e38)
    m_r[...] = jnp.maximum(m_r[...], part)

  return pl.pallas_call(
      body,
      grid=(_N // _B,),
      in_specs=[
          pl.BlockSpec((_B, 256), lambda i: (i, 0)),
          pl.BlockSpec((2, 256), lambda i: (0, 0)),
          pl.BlockSpec((1, 256), lambda i: (0, 0)),
          pl.BlockSpec((1, 256), lambda i: (0, 0)),
          pl.BlockSpec((256, 1), lambda i: (0, 0)),
          pl.BlockSpec((1, 1), lambda i: (0, 0)),
          pl.BlockSpec((_B, 1), lambda i: (i, 0)),
      ],
      out_specs=[pl.BlockSpec((_B, 256), lambda i: (i, 0)),
                 pl.BlockSpec((_B, 1), lambda i: (i, 0)),
                 pl.BlockSpec((1, _G), lambda i: (0, 0))],
      out_shape=[jax.ShapeDtypeStruct((_N, 256), jnp.float32),
                 jax.ShapeDtypeStruct((_N, 1), jnp.float32),
                 jax.ShapeDtypeStruct((1, _G), jnp.float32)],
  )(pre2, st2, g2.reshape(1, 256), be2.reshape(1, 256), Wg,
    bg.reshape(1, 1), batch2d)


def _tc_pool(d2, gate, m, batch2d):
  def body(d2_r, gate_r, m_r, bt_r, num_r, den_r):
    oh = (bt_r[...] == lax.broadcasted_iota(jnp.int32, (1, _G), 1)
          ).astype(jnp.float32)
    mnode = jnp.sum(oh * m_r[...], axis=1, keepdims=True)
    a = jnp.exp(gate_r[...] - mnode)

    @pl.when(pl.program_id(0) == 0)
    def _():
      num_r[...] = jnp.zeros_like(num_r)
      den_r[...] = jnp.zeros_like(den_r)
    num_r[...] += lax.dot_general(oh, a * d2_r[...],
                                  (((0,), (0,)), ((), ())),
                                  preferred_element_type=jnp.float32)
    den_r[...] += jnp.sum(oh * a, axis=0, keepdims=True)

  return pl.pallas_call(
      body,
      grid=(_N // _B,),
      in_specs=[
          pl.BlockSpec((_B, 256), lambda i: (i, 0)),
          pl.BlockSpec((_B, 1), lambda i: (i, 0)),
          pl.BlockSpec((1, _G), lambda i: (0, 0)),
          pl.BlockSpec((_B, 1), lambda i: (i, 0)),
      ],
      out_specs=[pl.BlockSpec((_G, 256), lambda i: (0, 0)),
                 pl.BlockSpec((1, _G), lambda i: (0, 0))],
      out_shape=[jax.ShapeDtypeStruct((_G, 256), jnp.float32),
                 jax.ShapeDtypeStruct((1, _G), jnp.float32)],
  )(d2, gate, m, batch2d)


def _tc_final(num, den):
  def body(num_r, den_r, out_r):
    recip = 1.0 / (den_r[...] + 1e-16)
    ii = lax.broadcasted_iota(jnp.int32, (_G, _G), 0)
    jj = lax.broadcasted_iota(jnp.int32, (_G, _G), 1)
    dmat = jnp.where(ii == jj, recip, 0.0)
    out_r[...] = jnp.dot(dmat, num_r[...], preferred_element_type=jnp.float32)

  return pl.pallas_call(
      body,
      out_shape=jax.ShapeDtypeStruct((_G, 256), jnp.float32),
  )(num, den)


def kernel(x, edge_index, batch, W_nf, b_nf, W1, b1, g1, be1,
           W2, b2, g2, be2, Wg, bg):
  src = edge_index[0]
  dst = edge_index[1]
  cnt = _sc_degree(dst)
  hs1, dinv = _tc_head(x, W_nf, b_nf, W1, cnt)
  agg1a, agg1b = _sc_edge_agg(hs1, hs1, src, dst, split_edges=True)
  pre1, st1 = _tc_pre1(agg1a, agg1b, hs1, dinv, b1)
  hs2a, hs2b = _tc_conv2_in(pre1, st1, g1, be1, W2, dinv)
  agg2a, agg2b = _sc_edge_agg(hs2a, hs2b, src, dst, split_edges=False)
  pre2, st2 = _tc_pre2(agg2a, agg2b, hs2a, hs2b, dinv, b2)
  d2, gate, m = _tc_gate(pre2, st2, g2, be2, Wg, bg, batch.reshape(_N, 1))
  num, den = _tc_pool(d2, gate, m, batch.reshape(_N, 1))
  return _tc_final(num, den)
